# initial kernel scaffold (unmeasured)
import jax
import jax.numpy as jnp
from jax import lax
from jax.experimental import pallas as pl
from jax.experimental.pallas import tpu as pltpu

N_DEV = 16
M_BLK = 256
K = 4096
K_BLK = 256
N = 2048


def kernel(x, w_mat):
    def body(x_ref, w_ref, out_ref, xbf_ref, wbf_ref, comm_ref,
             amax_snd_ref, amax_rcv_ref,
             send_sems, recv_sems, asnd_sems, arcv_sems):
        my = lax.axis_index("i")

        xbf_ref[...] = x_ref[...].astype(jnp.bfloat16)

        sends = []
        for t in range(1, N_DEV):
            dst = lax.rem(my + t, N_DEV)
            rdma = pltpu.make_async_remote_copy(
                src_ref=xbf_ref.at[pl.ds(dst * M_BLK, M_BLK), :],
                dst_ref=comm_ref.at[my],
                send_sem=send_sems.at[dst],
                recv_sem=recv_sems.at[my],
                device_id=(dst,),
                device_id_type=pl.DeviceIdType.MESH,
            )
            rdma.start()
            sends.append(rdma)

        wbf_ref[...] = w_ref[...].astype(jnp.bfloat16)

        out_ref[...] = jnp.dot(
            xbf_ref[pl.ds(my * M_BLK, M_BLK), :],
            wbf_ref[pl.ds(my * K_BLK, K_BLK), :],
            preferred_element_type=jnp.float32,
        )

        for t in range(1, N_DEV):
            j = lax.rem(my + N_DEV - t, N_DEV)
            recv = pltpu.make_async_remote_copy(
                src_ref=comm_ref.at[j],
                dst_ref=comm_ref.at[j],
                send_sem=send_sems.at[j],
                recv_sem=recv_sems.at[j],
                device_id=(j,),
                device_id_type=pl.DeviceIdType.MESH,
            )
            recv.wait_recv()
            out_ref[...] += jnp.dot(
                comm_ref[j],
                wbf_ref[pl.ds(j * K_BLK, K_BLK), :],
                preferred_element_type=jnp.float32,
            )

        for rdma in sends:
            rdma.wait_send()

        amax = jnp.max(jnp.abs(out_ref[...]))
        for r in range(4):
            partner = lax.bitwise_xor(my, 1 << r)
            amax_snd_ref[r, :, :] = jnp.full((8, 128), amax, jnp.float32)
            bf = pltpu.make_async_remote_copy(
                src_ref=amax_snd_ref.at[r],
                dst_ref=amax_rcv_ref.at[r],
                send_sem=asnd_sems.at[r],
                recv_sem=arcv_sems.at[r],
                device_id=(partner,),
                device_id_type=pl.DeviceIdType.MESH,
            )
            bf.start()
            bf.wait()
            amax = jnp.maximum(amax, amax_rcv_ref[r, 0, 0])

        scale = amax / 448.0
        q = (out_ref[...] / scale).astype(jnp.float8_e4m3fn)
        out_ref[...] = q.astype(jnp.float32) * scale

    return pl.pallas_call(
        body,
        out_shape=jax.ShapeDtypeStruct((M_BLK, N), jnp.float32),
        in_specs=[
            pl.BlockSpec(memory_space=pltpu.VMEM),
            pl.BlockSpec(memory_space=pltpu.VMEM),
        ],
        out_specs=pl.BlockSpec(memory_space=pltpu.VMEM),
        scratch_shapes=[
            pltpu.VMEM((K, K_BLK), jnp.bfloat16),
            pltpu.VMEM((K, N), jnp.bfloat16),
            pltpu.VMEM((N_DEV, M_BLK, K_BLK), jnp.bfloat16),
            pltpu.VMEM((4, 8, 128), jnp.float32),
            pltpu.VMEM((4, 8, 128), jnp.float32),
            pltpu.SemaphoreType.DMA((N_DEV,)),
            pltpu.SemaphoreType.DMA((N_DEV,)),
            pltpu.SemaphoreType.DMA((4,)),
            pltpu.SemaphoreType.DMA((4,)),
        ],
        compiler_params=pltpu.CompilerParams(collective_id=0),
    )(x, w_mat)


# baseline (device time: 60113 ns/iter reference)
import jax
import jax.numpy as jnp
from jax import lax
from jax.experimental import pallas as pl
from jax.experimental.pallas import tpu as pltpu

N_DEV = 16
M_BLK = 256
K = 4096
K_BLK = 256
N = 2048


def kernel(x, w_mat):
    def body(x_ref, w_ref, out_ref, xbf_ref, wbf_ref, comm_ref,
             amax_snd_ref, amax_rcv_ref,
             send_sems, recv_sems, asnd_sems, arcv_sems):
        my = lax.axis_index("i")

        barrier_sem = pltpu.get_barrier_semaphore()
        for t in range(1, N_DEV):
            peer = lax.rem(my + t, N_DEV)
            pl.semaphore_signal(
                barrier_sem, inc=1,
                device_id=(peer,), device_id_type=pl.DeviceIdType.MESH,
            )
        pl.semaphore_wait(barrier_sem, N_DEV - 1)

        xbf_ref[...] = x_ref[...].astype(jnp.bfloat16)

        sends = []
        for t in range(1, N_DEV):
            dst = lax.rem(my + t, N_DEV)
            rdma = pltpu.make_async_remote_copy(
                src_ref=xbf_ref.at[pl.ds(dst * M_BLK, M_BLK), :],
                dst_ref=comm_ref.at[my],
                send_sem=send_sems.at[dst],
                recv_sem=recv_sems.at[my],
                device_id=(dst,),
                device_id_type=pl.DeviceIdType.MESH,
            )
            rdma.start()
            sends.append(rdma)

        wbf_ref[...] = w_ref[...].astype(jnp.bfloat16)

        out_ref[...] = jnp.dot(
            xbf_ref[pl.ds(my * M_BLK, M_BLK), :],
            wbf_ref[pl.ds(my * K_BLK, K_BLK), :],
            preferred_element_type=jnp.float32,
        )

        for t in range(1, N_DEV):
            j = lax.rem(my + N_DEV - t, N_DEV)
            recv = pltpu.make_async_remote_copy(
                src_ref=comm_ref.at[j],
                dst_ref=comm_ref.at[j],
                send_sem=send_sems.at[j],
                recv_sem=recv_sems.at[j],
                device_id=(j,),
                device_id_type=pl.DeviceIdType.MESH,
            )
            recv.wait_recv()
            out_ref[...] += jnp.dot(
                comm_ref[j],
                wbf_ref[pl.ds(j * K_BLK, K_BLK), :],
                preferred_element_type=jnp.float32,
            )

        for rdma in sends:
            rdma.wait_send()

        amax = jnp.max(jnp.abs(out_ref[...]))
        for r in range(4):
            partner = lax.bitwise_xor(my, 1 << r)
            amax_snd_ref[r, :, :] = jnp.full((8, 128), amax, jnp.float32)
            bf = pltpu.make_async_remote_copy(
                src_ref=amax_snd_ref.at[r],
                dst_ref=amax_rcv_ref.at[r],
                send_sem=asnd_sems.at[r],
                recv_sem=arcv_sems.at[r],
                device_id=(partner,),
                device_id_type=pl.DeviceIdType.MESH,
            )
            bf.start()
            bf.wait()
            amax = jnp.maximum(amax, amax_rcv_ref[r, 0, 0])

        scale = amax / 448.0
        q = (out_ref[...] / scale).astype(jnp.float8_e4m3fn)
        out_ref[...] = q.astype(jnp.float32) * scale

    return pl.pallas_call(
        body,
        out_shape=jax.ShapeDtypeStruct((M_BLK, N), jnp.float32),
        in_specs=[
            pl.BlockSpec(memory_space=pltpu.VMEM),
            pl.BlockSpec(memory_space=pltpu.VMEM),
        ],
        out_specs=pl.BlockSpec(memory_space=pltpu.VMEM),
        scratch_shapes=[
            pltpu.VMEM((K, K_BLK), jnp.bfloat16),
            pltpu.VMEM((K, N), jnp.bfloat16),
            pltpu.VMEM((N_DEV, M_BLK, K_BLK), jnp.bfloat16),
            pltpu.VMEM((4, 8, 128), jnp.float32),
            pltpu.VMEM((4, 8, 128), jnp.float32),
            pltpu.SemaphoreType.DMA((N_DEV,)),
            pltpu.SemaphoreType.DMA((N_DEV,)),
            pltpu.SemaphoreType.DMA((4,)),
            pltpu.SemaphoreType.DMA((4,)),
        ],
        compiler_params=pltpu.CompilerParams(
            vmem_limit_bytes=100 * 1024 * 1024,
            collective_id=0,
        ),
    )(x, w_mat)
